# baseline (speedup 1.0000x reference)
import jax
import jax.numpy as jnp
from jax import lax
from jax.experimental import pallas as pl
from jax.experimental.pallas import tpu as pltpu
from jax.experimental.pallas import tpu_sc as plsc

NC = 2
NS = 16
NW = NC * NS

F32 = jnp.float32
BF16 = jnp.bfloat16

_SC_PARAMS = pltpu.CompilerParams(use_tc_tiling_on_sc=True)


def _largest_chunk(per):
    best = 8
    for c in range(8, 129, 8):
        if per % c == 0:
            best = c
    return best


def _edge_block(E):
    for b in range(4096, 7, -8):
        if E % b == 0:
            return b
    return E


def _sc_gather(ta, tbc, row, col):
    E = row.shape[0]
    per = E // NW
    ch = _largest_chunk(per)
    nchunks = per // ch

    mesh = plsc.VectorSubcoreMesh(core_axis_name="c", subcore_axis_name="s",
                                  num_cores=NC, num_subcores=NS)

    def body(ta_h, tbc_h, row_h, col_h, g_h,
             rowi, coli, ca, cb, sg0, sg1, sw0, sw1):
        c = lax.axis_index("c")
        s = lax.axis_index("s")
        wid = s * NC + c
        start = pl.multiple_of(wid * per, ch)
        pltpu.sync_copy(row_h.at[pl.ds(start, per)], rowi)
        pltpu.sync_copy(col_h.at[pl.ds(start, per)], coli)

        sg = (sg0, sg1)
        sw = (sw0, sw1)

        def issue(b, t, first):
            if not first:
                pltpu.make_async_copy(cb.at[b], g_h.at[pl.ds(start, ch)],
                                      sw[b]).wait()
            ro = rowi.at[pl.ds(t * ch, ch)]
            co = coli.at[pl.ds(t * ch, ch)]
            pltpu.async_copy(ta_h.at[ro], ca.at[b], sg[b])
            pltpu.async_copy(tbc_h.at[co], cb.at[b], sg[b])

        def finish(b, t):
            base = pl.multiple_of(start + t * ch, ch)
            ro = rowi.at[pl.ds(0, ch)]
            pltpu.make_async_copy(ta_h.at[ro], ca.at[b], sg[b]).wait()
            pltpu.make_async_copy(ta_h.at[ro], cb.at[b], sg[b]).wait()

            def addrow(r, carry):
                for k in range(8):
                    sl = pl.ds(16 * k, 16)
                    cb[b, r, sl] = cb[b, r, sl] + ca[b, r, sl]
                return carry

            lax.fori_loop(0, ch, addrow, 0, unroll=8)
            pltpu.async_copy(cb.at[b], g_h.at[pl.ds(base, ch)], sw[b])

        issue(0, 0, True)
        if nchunks > 1:
            issue(1, 1, True)

        def pair(tp, carry):
            t0 = 2 * tp
            t1 = t0 + 1
            finish(0, t0)
            issue(0, t0 + 2, False)
            finish(1, t1)

            @pl.when(t1 + 2 < nchunks)
            def _():
                issue(1, t1 + 2, False)

            return carry

        if nchunks % 2 == 1:
            lax.fori_loop(0, (nchunks - 1) // 2, pair, 0)
            finish(0, nchunks - 1)
        else:
            lax.fori_loop(0, (nchunks - 2) // 2, pair, 0)
            finish(0, nchunks - 2)
            finish(1, nchunks - 1)
        pltpu.make_async_copy(cb.at[0], g_h.at[pl.ds(start, ch)], sw[0]).wait()
        if nchunks > 1:
            pltpu.make_async_copy(cb.at[1], g_h.at[pl.ds(start, ch)],
                                  sw[1]).wait()

    call = pl.kernel(
        body,
        out_type=jax.ShapeDtypeStruct((E, 128), F32),
        mesh=mesh,
        scratch_types=[
            pltpu.VMEM((per,), jnp.int32),
            pltpu.VMEM((per,), jnp.int32),
            pltpu.VMEM((2, ch, 128), F32),
            pltpu.VMEM((2, ch, 128), F32),
            pltpu.SemaphoreType.DMA,
            pltpu.SemaphoreType.DMA,
            pltpu.SemaphoreType.DMA,
            pltpu.SemaphoreType.DMA,
        ],
        compiler_params=_SC_PARAMS,
    )
    return call(ta, tbc, row, col)


def _sc_scatter(vals_list, row3_list, prev):
    k = len(vals_list)
    infos = []
    for v, r3 in zip(vals_list, row3_list):
        infos.append((v.shape[0] // NW, r3.shape[2], r3.shape[1]))
    chmax = max(ch for _, ch, _ in infos)
    npad = prev.shape[1]
    rpt = npad // NS

    mesh = plsc.VectorSubcoreMesh(core_axis_name="c", subcore_axis_name="s",
                                  num_cores=NC, num_subcores=NS)

    def body(*refs):
        vals_hs = refs[0:k]
        row3_hs = refs[k:2 * k]
        prev_h = refs[2 * k]
        sum_h = refs[2 * k + 1]
        idxvs = refs[2 * k + 2:3 * k + 2]
        valvs = refs[3 * k + 2:4 * k + 2]
        ssum, sl0, sl1, ss0, ss1 = refs[4 * k + 2:]
        c = lax.axis_index("c")
        s = lax.axis_index("s")
        wid = s * NC + c
        soff = pl.multiple_of(s * rpt, 8)

        pltpu.sync_copy(prev_h.at[c, pl.ds(soff, rpt)],
                        ssum.at[pl.ds(soff, rpt)])
        plsc.subcore_barrier()

        sl = (sl0, sl1)
        ss = (ss0, ss1)

        for p in range(k):
            per, ch, nchunks = infos[p]
            vals_h = vals_hs[p]
            idxv = idxvs[p]
            valv = valvs[p]
            start = pl.multiple_of(wid * per, ch)
            pltpu.sync_copy(row3_hs[p].at[wid], idxv)

            def vbuf(b):
                return valv.at[b]

            def wait_scat(b):
                pltpu.make_async_copy(vbuf(b), ssum.at[idxv.at[0]],
                                      ss[b]).wait()

            def load(b, t, first):
                if not first:
                    wait_scat(b)
                pltpu.async_copy(
                    vals_h.at[pl.ds(pl.multiple_of(start + t * ch, ch), ch)],
                    vbuf(b), sl[b])

            def scat(b, t):
                pltpu.make_async_copy(vals_h.at[pl.ds(start, ch)],
                                      vbuf(b), sl[b]).wait()
                pltpu.async_copy(vbuf(b), ssum.at[idxv.at[t]], ss[b],
                                 add=True)

            load(0, 0, True)
            if nchunks > 1:
                load(1, 1, True)

            def pair(tp, carry):
                t0 = 2 * tp
                t1 = t0 + 1
                scat(0, t0)
                load(0, t0 + 2, False)
                scat(1, t1)

                @pl.when(t1 + 2 < nchunks)
                def _():
                    load(1, t1 + 2, False)

                return carry

            if nchunks % 2 == 1:
                lax.fori_loop(0, (nchunks - 1) // 2, pair, 0)
                scat(0, nchunks - 1)
            else:
                lax.fori_loop(0, (nchunks - 2) // 2, pair, 0)
                scat(0, nchunks - 2)
                scat(1, nchunks - 1)
            wait_scat(0)
            if nchunks > 1:
                wait_scat(1)
        plsc.subcore_barrier()

        pltpu.sync_copy(ssum.at[pl.ds(soff, rpt)],
                        sum_h.at[c, pl.ds(soff, rpt)])

    call = pl.kernel(
        body,
        out_type=jax.ShapeDtypeStruct((NC, npad, 128), F32),
        mesh=mesh,
        scratch_types=(
            [pltpu.VMEM((nch, ch), jnp.int32) for _, ch, nch in infos]
            + [pltpu.VMEM((2, ch, 128), F32) for _, ch, _ in infos] + [
                pltpu.VMEM_SHARED((npad, 128), F32),
                pltpu.SemaphoreType.DMA,
                pltpu.SemaphoreType.DMA,
                pltpu.SemaphoreType.DMA,
                pltpu.SemaphoreType.DMA,
            ]),
        compiler_params=_SC_PARAMS,
    )
    return call(*vals_list, *row3_list, prev)


def _full(shape):
    return pl.BlockSpec(shape, lambda i: (0,) * len(shape))


def _node_pre(x, wcat):
    n, d = x.shape
    bn = 2000 if n % 2000 == 0 else n

    def body(x_ref, w_ref, ta_ref, tbc_ref):
        t = jnp.dot(x_ref[...], w_ref[...], preferred_element_type=F32)
        ta_ref[...] = jnp.concatenate(
            [t[:, 0:64], jnp.zeros(t[:, 0:64].shape, F32)], axis=1)
        tbc_ref[...] = t[:, 64:192]

    o = jax.ShapeDtypeStruct((n, 128), F32)
    return pl.pallas_call(
        body,
        grid=(n // bn,),
        in_specs=[pl.BlockSpec((bn, d), lambda i: (i, 0)), _full(wcat.shape)],
        out_specs=[pl.BlockSpec((bn, 128), lambda i: (i, 0))] * 2,
        out_shape=[o, o],
    )(x, wcat)


def _edge1(gcat, ea, wec, be0, w_e1, be1, wla, bn0, w_n1, bn1):
    E = gcat.shape[0]
    be = _edge_block(E)

    def body(g_ref, ea_ref, wec_ref, be0_ref, we1_ref, be1_ref,
             wla_ref, bn0_ref, wn1_ref, bn1_ref, la_ref, m_ref):
        g = g_ref[...].astype(F32)
        eaw = lax.dot_general(ea_ref[...], wec_ref[...],
                              (((0,), (0,)), ((), ())),
                              preferred_element_type=F32)
        h1 = jnp.maximum(g[:, 0:64] + eaw + be0_ref[...], 0.0)
        la = jnp.dot(h1, we1_ref[...], preferred_element_type=F32) + be1_ref[...]
        la_ref[...] = la
        h2 = jnp.maximum(
            g[:, 64:128]
            + jnp.dot(la, wla_ref[...], preferred_element_type=F32)
            + bn0_ref[...], 0.0)
        m1 = jnp.dot(h2, wn1_ref[...], preferred_element_type=F32) + bn1_ref[...]
        lane = lax.broadcasted_iota(jnp.int32, (m1.shape[0], 64), 1)
        cnt = jnp.where(lane == 0, 1.0, 0.0).astype(F32)
        m_ref[...] = jnp.concatenate([m1, cnt], axis=1)

    eb64 = pl.BlockSpec((be, 64), lambda i: (i, 0))
    eb128 = pl.BlockSpec((be, 128), lambda i: (i, 0))
    return pl.pallas_call(
        body,
        grid=(E // be,),
        in_specs=[eb128, pl.BlockSpec((16, be), lambda i: (0, i)),
                  _full(wec.shape), _full(be0.shape), _full(w_e1.shape),
                  _full(be1.shape), _full(wla.shape), _full(bn0.shape),
                  _full(w_n1.shape), _full(bn1.shape)],
        out_specs=[eb64, eb128],
        out_shape=[jax.ShapeDtypeStruct((E, 64), F32),
                   jax.ShapeDtypeStruct((E, 128), F32)],
    )(gcat, ea, wec, be0, w_e1, be1, wla, bn0, w_n1, bn1)


def _node1(sums, x, w0x, w0a, b0, w1, b1, wpost):
    n = x.shape[0]
    bn = 2000 if n % 2000 == 0 else n

    def body(s_ref, x_ref, w0x_ref, w0a_ref, b0_ref, w1_ref, b1_ref,
             wp_ref, nf_ref, t2a_ref, t2bc_ref, rden_ref):
        stot = s_ref[0] + s_ref[1]
        rden = 1.0 / jnp.maximum(stot[:, 64:65], 1.0)
        agg = stot[:, 0:64] * rden
        h = jnp.maximum(
            jnp.dot(x_ref[...], w0x_ref[...], preferred_element_type=F32)
            + jnp.dot(agg, w0a_ref[...], preferred_element_type=F32)
            + b0_ref[...], 0.0)
        nf = jnp.maximum(
            jnp.dot(h, w1_ref[...], preferred_element_type=F32) + b1_ref[...], 0.0)
        nf_ref[...] = nf
        rden_ref[...] = rden
        t = jnp.dot(nf, wp_ref[...], preferred_element_type=F32)
        t2a_ref[...] = jnp.concatenate(
            [t[:, 0:64], jnp.zeros(t[:, 0:64].shape, F32)], axis=1)
        t2bc_ref[...] = t[:, 64:192]

    return pl.pallas_call(
        body,
        grid=(n // bn,),
        in_specs=[pl.BlockSpec((NC, bn, 128), lambda i: (0, i, 0)),
                  pl.BlockSpec((bn, 128), lambda i: (i, 0)),
                  _full(w0x.shape), _full(w0a.shape), _full(b0.shape),
                  _full(w1.shape), _full(b1.shape), _full(wpost.shape)],
        out_specs=[pl.BlockSpec((bn, 64), lambda i: (i, 0)),
                   pl.BlockSpec((bn, 128), lambda i: (i, 0)),
                   pl.BlockSpec((bn, 128), lambda i: (i, 0)),
                   pl.BlockSpec((bn, 1), lambda i: (i, 0))],
        out_shape=[jax.ShapeDtypeStruct((n, 64), F32),
                   jax.ShapeDtypeStruct((n, 128), F32),
                   jax.ShapeDtypeStruct((n, 128), F32),
                   jax.ShapeDtypeStruct((n, 1), F32)],
    )(sums, x, w0x, w0a, b0, w1, b1, wpost)


def _edge2(gcat, la, wla2, be0, w_e2, be1, wn2a, bn0, w_n2, bn1):
    E = gcat.shape[0]
    be = _edge_block(E)

    def body(g_ref, la_ref, wla2_ref, be0_ref, we2_ref, be1_ref,
             wn2a_ref, bn0_ref, wn2_ref, bn1_ref, m_ref):
        g = g_ref[...].astype(F32)
        h3 = jnp.maximum(
            g[:, 0:64]
            + jnp.dot(la_ref[...], wla2_ref[...], preferred_element_type=F32)
            + be0_ref[...], 0.0)
        la2 = jnp.dot(h3, we2_ref[...], preferred_element_type=F32) + be1_ref[...]
        h4 = jnp.maximum(
            g[:, 64:128]
            + jnp.dot(la2, wn2a_ref[...], preferred_element_type=F32)
            + bn0_ref[...], 0.0)
        m2 = jnp.dot(h4, wn2_ref[...], preferred_element_type=F32) + bn1_ref[...]
        m_ref[...] = jnp.concatenate([m2, jnp.zeros(m2.shape, F32)], axis=1)

    eb64 = pl.BlockSpec((be, 64), lambda i: (i, 0))
    eb128 = pl.BlockSpec((be, 128), lambda i: (i, 0))
    return pl.pallas_call(
        body,
        grid=(E // be,),
        in_specs=[eb128, eb64,
                  _full(wla2.shape), _full(be0.shape), _full(w_e2.shape),
                  _full(be1.shape), _full(wn2a.shape), _full(bn0.shape),
                  _full(w_n2.shape), _full(bn1.shape)],
        out_specs=eb128,
        out_shape=jax.ShapeDtypeStruct((E, 128), F32),
    )(gcat, la, wla2, be0, w_e2, be1, wn2a, bn0, w_n2, bn1)


def _node2(sums, rden, nf, w0n, w0a, b0, w1, b1):
    n = nf.shape[0]
    bn = 2000 if n % 2000 == 0 else n

    def body(s_ref, rd_ref, nf_ref, w0n_ref, w0a_ref, b0_ref,
             w1_ref, b1_ref, out_ref):
        stot = s_ref[0] + s_ref[1]
        agg = stot[:, 0:64] * rd_ref[...]
        h = jnp.maximum(
            jnp.dot(nf_ref[...], w0n_ref[...], preferred_element_type=F32)
            + jnp.dot(agg, w0a_ref[...], preferred_element_type=F32)
            + b0_ref[...], 0.0)
        out_ref[...] = jnp.dot(h, w1_ref[...], preferred_element_type=F32) + b1_ref[...]

    return pl.pallas_call(
        body,
        grid=(n // bn,),
        in_specs=[pl.BlockSpec((NC, bn, 128), lambda i: (0, i, 0)),
                  pl.BlockSpec((bn, 1), lambda i: (i, 0)),
                  pl.BlockSpec((bn, 64), lambda i: (i, 0)),
                  _full(w0n.shape), _full(w0a.shape), _full(b0.shape),
                  _full(w1.shape), _full(b1.shape)],
        out_specs=pl.BlockSpec((bn, 1), lambda i: (i, 0)),
        out_shape=jax.ShapeDtypeStruct((n, 1), F32),
    )(sums, rden, nf, w0n, w0a, b0, w1, b1)


def kernel(x, edge_index, edge_attr,
           e1_W0, e1_b0, e1_W1, e1_b1,
           n1a_W0, n1a_b0, n1a_W1, n1a_b1,
           n1u_W0, n1u_b0, n1u_W1, n1u_b1,
           e2_W0, e2_b0, e2_W1, e2_b1,
           n2a_W0, n2a_b0, n2a_W1, n2a_b1,
           n2u_W0, n2u_b0, n2u_W1, n2u_b1):
    n, d = x.shape
    E = edge_index.shape[1]
    npad = -(-n // 128) * 128
    row = edge_index[0]
    col = edge_index[1]
    eaT = edge_attr.T

    if E % (NW * 8) == 0 and E > 3 * 81920 and (E - 3 * 81920) % (NW * 8) == 0:
        parts = [81920, 81920, 81920, E - 3 * 81920]
    else:
        parts = [E]
    offs = [sum(parts[:i]) for i in range(len(parts))]

    rows = [lax.slice(row, (o,), (o + q,)) for o, q in zip(offs, parts)]
    cols = [lax.slice(col, (o,), (o + q,)) for o, q in zip(offs, parts)]
    row3s = []
    for r, q in zip(rows, parts):
        per = q // NW
        ch = _largest_chunk(per)
        row3s.append(r.reshape(NW, per // ch, ch))
    eaTs = [lax.slice(eaT, (0, o), (eaT.shape[0], o + q))
            for o, q in zip(offs, parts)]

    r1 = lambda b: b.reshape(1, -1)

    wcat1 = jnp.concatenate([e1_W0[:d], e1_W0[d:2 * d], n1a_W0[:d]], axis=1)
    ta, tbc = _node_pre(x, wcat1)

    e1_w = (e1_W0[2 * d:], r1(e1_b0), e1_W1, r1(e1_b1),
            n1a_W0[d:], r1(n1a_b0), n1a_W1, r1(n1a_b1))
    zero_part = jnp.zeros((NC, npad, 128), F32)

    gcats = [_sc_gather(ta, tbc, r, c) for r, c in zip(rows, cols)]
    las, mcats = [], []
    for g, ea in zip(gcats, eaTs):
        la_i, m_i = _edge1(g, ea, *e1_w)
        las.append(la_i)
        mcats.append(m_i)
    s1 = zero_part
    for m_i, r3 in zip(mcats, row3s):
        s1 = _sc_scatter([m_i], [r3], s1)

    wpost = jnp.concatenate([e2_W0[:64], e2_W0[64:128], n2a_W0[:64]], axis=1)
    nf, t2a, t2bc, rden = _node1(s1, x,
                                 n1u_W0[:d], n1u_W0[d:], r1(n1u_b0),
                                 n1u_W1, r1(n1u_b1), wpost)

    e2_w = (e2_W0[128:], r1(e2_b0), e2_W1, r1(e2_b1),
            n2a_W0[64:], r1(n2a_b0), n2a_W1, r1(n2a_b1))
    g2cats = [_sc_gather(t2a, t2bc, r, c) for r, c in zip(rows, cols)]
    m2cats = [_edge2(g, la_i, *e2_w)
              for g, la_i in zip(g2cats, las)]
    s2 = zero_part
    for m_i, r3 in zip(m2cats, row3s):
        s2 = _sc_scatter([m_i], [r3], s2)

    return _node2(s2, rden, nf,
                  n2u_W0[:64], n2u_W0[64:], r1(n2u_b0),
                  n2u_W1, r1(n2u_b1))


# speedup vs baseline: 1.1743x; 1.1743x over previous
import jax
import jax.numpy as jnp
from jax import lax
from jax.experimental import pallas as pl
from jax.experimental.pallas import tpu as pltpu
from jax.experimental.pallas import tpu_sc as plsc

NC = 2
NS = 16
NW = NC * NS

F32 = jnp.float32
BF16 = jnp.bfloat16

_SC_PARAMS = pltpu.CompilerParams(use_tc_tiling_on_sc=True)


def _largest_chunk(per):
    best = 8
    for c in range(8, 129, 8):
        if per % c == 0:
            best = c
    return best


def _edge_block(E):
    for b in range(4096, 7, -8):
        if E % b == 0:
            return b
    return E


def _sc_gather(ta, tbc, row, col):
    E = row.shape[0]
    per = E // NW
    ch = _largest_chunk(per)
    nchunks = per // ch

    mesh = plsc.VectorSubcoreMesh(core_axis_name="c", subcore_axis_name="s",
                                  num_cores=NC, num_subcores=NS)

    def body(ta_h, tbc_h, row_h, col_h, g_h,
             rowi, coli, ca, cb, sg0, sg1, sw0, sw1):
        c = lax.axis_index("c")
        s = lax.axis_index("s")
        wid = s * NC + c
        start = pl.multiple_of(wid * per, ch)
        pltpu.sync_copy(row_h.at[pl.ds(start, per)], rowi)
        pltpu.sync_copy(col_h.at[pl.ds(start, per)], coli)

        sg = (sg0, sg1)
        sw = (sw0, sw1)

        def issue(b, t, first):
            if not first:
                pltpu.make_async_copy(cb.at[b], g_h.at[pl.ds(start, ch)],
                                      sw[b]).wait()
            ro = rowi.at[pl.ds(t * ch, ch)]
            co = coli.at[pl.ds(t * ch, ch)]
            pltpu.async_copy(ta_h.at[ro], ca.at[b], sg[b])
            pltpu.async_copy(tbc_h.at[co], cb.at[b], sg[b])

        def finish(b, t):
            base = pl.multiple_of(start + t * ch, ch)
            ro = rowi.at[pl.ds(0, ch)]
            pltpu.make_async_copy(ta_h.at[ro], ca.at[b], sg[b]).wait()
            pltpu.make_async_copy(ta_h.at[ro], cb.at[b], sg[b]).wait()

            def addrow(r, carry):
                for k in range(4):
                    sl = pl.ds(16 * k, 16)
                    cb[b, r, sl] = cb[b, r, sl] + ca[b, r, sl]
                return carry

            lax.fori_loop(0, ch, addrow, 0, unroll=8)
            pltpu.async_copy(cb.at[b], g_h.at[pl.ds(base, ch)], sw[b])

        issue(0, 0, True)
        if nchunks > 1:
            issue(1, 1, True)

        def pair(tp, carry):
            t0 = 2 * tp
            t1 = t0 + 1
            finish(0, t0)
            issue(0, t0 + 2, False)
            finish(1, t1)

            @pl.when(t1 + 2 < nchunks)
            def _():
                issue(1, t1 + 2, False)

            return carry

        if nchunks % 2 == 1:
            lax.fori_loop(0, (nchunks - 1) // 2, pair, 0)
            finish(0, nchunks - 1)
        else:
            lax.fori_loop(0, (nchunks - 2) // 2, pair, 0)
            finish(0, nchunks - 2)
            finish(1, nchunks - 1)
        pltpu.make_async_copy(cb.at[0], g_h.at[pl.ds(start, ch)], sw[0]).wait()
        if nchunks > 1:
            pltpu.make_async_copy(cb.at[1], g_h.at[pl.ds(start, ch)],
                                  sw[1]).wait()

    call = pl.kernel(
        body,
        out_type=jax.ShapeDtypeStruct((E, 128), F32),
        mesh=mesh,
        scratch_types=[
            pltpu.VMEM((per,), jnp.int32),
            pltpu.VMEM((per,), jnp.int32),
            pltpu.VMEM((2, ch, 128), F32),
            pltpu.VMEM((2, ch, 128), F32),
            pltpu.SemaphoreType.DMA,
            pltpu.SemaphoreType.DMA,
            pltpu.SemaphoreType.DMA,
            pltpu.SemaphoreType.DMA,
        ],
        compiler_params=_SC_PARAMS,
    )
    return call(ta, tbc, row, col)


def _sc_scatter(vals_list, row3_list, prev):
    k = len(vals_list)
    infos = []
    for v, r3 in zip(vals_list, row3_list):
        infos.append((v.shape[0] // NW, r3.shape[2], r3.shape[1]))
    chmax = max(ch for _, ch, _ in infos)
    npad = prev.shape[1]
    rpt = npad // NS

    mesh = plsc.VectorSubcoreMesh(core_axis_name="c", subcore_axis_name="s",
                                  num_cores=NC, num_subcores=NS)

    def body(*refs):
        vals_hs = refs[0:k]
        row3_hs = refs[k:2 * k]
        prev_h = refs[2 * k]
        sum_h = refs[2 * k + 1]
        idxvs = refs[2 * k + 2:3 * k + 2]
        valvs = refs[3 * k + 2:4 * k + 2]
        ssum, sl0, sl1, ss0, ss1 = refs[4 * k + 2:]
        c = lax.axis_index("c")
        s = lax.axis_index("s")
        wid = s * NC + c
        soff = pl.multiple_of(s * rpt, 8)

        pltpu.sync_copy(prev_h.at[c, pl.ds(soff, rpt)],
                        ssum.at[pl.ds(soff, rpt)])
        plsc.subcore_barrier()

        sl = (sl0, sl1)
        ss = (ss0, ss1)

        for p in range(k):
            per, ch, nchunks = infos[p]
            vals_h = vals_hs[p]
            idxv = idxvs[p]
            valv = valvs[p]
            start = pl.multiple_of(wid * per, ch)
            pltpu.sync_copy(row3_hs[p].at[wid], idxv)

            def vbuf(b):
                return valv.at[b]

            def wait_scat(b):
                pltpu.make_async_copy(vbuf(b), ssum.at[idxv.at[0]],
                                      ss[b]).wait()

            def load(b, t, first):
                if not first:
                    wait_scat(b)
                pltpu.async_copy(
                    vals_h.at[pl.ds(pl.multiple_of(start + t * ch, ch), ch)],
                    vbuf(b), sl[b])

            def scat(b, t):
                pltpu.make_async_copy(vals_h.at[pl.ds(start, ch)],
                                      vbuf(b), sl[b]).wait()
                pltpu.async_copy(vbuf(b), ssum.at[idxv.at[t]], ss[b],
                                 add=True)

            load(0, 0, True)
            if nchunks > 1:
                load(1, 1, True)

            def pair(tp, carry):
                t0 = 2 * tp
                t1 = t0 + 1
                scat(0, t0)
                load(0, t0 + 2, False)
                scat(1, t1)

                @pl.when(t1 + 2 < nchunks)
                def _():
                    load(1, t1 + 2, False)

                return carry

            if nchunks % 2 == 1:
                lax.fori_loop(0, (nchunks - 1) // 2, pair, 0)
                scat(0, nchunks - 1)
            else:
                lax.fori_loop(0, (nchunks - 2) // 2, pair, 0)
                scat(0, nchunks - 2)
                scat(1, nchunks - 1)
            wait_scat(0)
            if nchunks > 1:
                wait_scat(1)
        plsc.subcore_barrier()

        pltpu.sync_copy(ssum.at[pl.ds(soff, rpt)],
                        sum_h.at[c, pl.ds(soff, rpt)])

    call = pl.kernel(
        body,
        out_type=jax.ShapeDtypeStruct((NC, npad, 128), F32),
        mesh=mesh,
        scratch_types=(
            [pltpu.VMEM((nch, ch), jnp.int32) for _, ch, nch in infos]
            + [pltpu.VMEM((2, ch, 128), F32) for _, ch, _ in infos] + [
                pltpu.VMEM_SHARED((npad, 128), F32),
                pltpu.SemaphoreType.DMA,
                pltpu.SemaphoreType.DMA,
                pltpu.SemaphoreType.DMA,
                pltpu.SemaphoreType.DMA,
            ]),
        compiler_params=_SC_PARAMS,
    )
    return call(*vals_list, *row3_list, prev)


def _full(shape):
    return pl.BlockSpec(shape, lambda i: (0,) * len(shape))


def _node_pre(x, wcat):
    n, d = x.shape
    bn = 2000 if n % 2000 == 0 else n

    def body(x_ref, w_ref, ta_ref, tbc_ref):
        t = jnp.dot(x_ref[...], w_ref[...], preferred_element_type=F32)
        ta_ref[...] = jnp.concatenate(
            [t[:, 0:64], jnp.zeros(t[:, 0:64].shape, F32)], axis=1)
        tbc_ref[...] = t[:, 64:192]

    o = jax.ShapeDtypeStruct((n, 128), F32)
    return pl.pallas_call(
        body,
        grid=(n // bn,),
        in_specs=[pl.BlockSpec((bn, d), lambda i: (i, 0)), _full(wcat.shape)],
        out_specs=[pl.BlockSpec((bn, 128), lambda i: (i, 0))] * 2,
        out_shape=[o, o],
    )(x, wcat)


def _edge1(gcat, ea, wec, be0, w_e1, be1, wla, bn0, w_n1, bn1):
    E = gcat.shape[0]
    be = _edge_block(E)

    def body(g_ref, ea_ref, wec_ref, be0_ref, we1_ref, be1_ref,
             wla_ref, bn0_ref, wn1_ref, bn1_ref, la_ref, m_ref):
        g = g_ref[...].astype(F32)
        eaw = lax.dot_general(ea_ref[...], wec_ref[...],
                              (((0,), (0,)), ((), ())),
                              preferred_element_type=F32)
        h1 = jnp.maximum(g[:, 0:64] + eaw + be0_ref[...], 0.0)
        la = jnp.dot(h1, we1_ref[...], preferred_element_type=F32) + be1_ref[...]
        la_ref[...] = la
        h2 = jnp.maximum(
            g[:, 64:128]
            + jnp.dot(la, wla_ref[...], preferred_element_type=F32)
            + bn0_ref[...], 0.0)
        m1 = jnp.dot(h2, wn1_ref[...], preferred_element_type=F32) + bn1_ref[...]
        lane = lax.broadcasted_iota(jnp.int32, (m1.shape[0], 64), 1)
        cnt = jnp.where(lane == 0, 1.0, 0.0).astype(F32)
        m_ref[...] = jnp.concatenate([m1, cnt], axis=1)

    eb64 = pl.BlockSpec((be, 64), lambda i: (i, 0))
    eb128 = pl.BlockSpec((be, 128), lambda i: (i, 0))
    return pl.pallas_call(
        body,
        grid=(E // be,),
        in_specs=[eb128, pl.BlockSpec((16, be), lambda i: (0, i)),
                  _full(wec.shape), _full(be0.shape), _full(w_e1.shape),
                  _full(be1.shape), _full(wla.shape), _full(bn0.shape),
                  _full(w_n1.shape), _full(bn1.shape)],
        out_specs=[eb64, eb128],
        out_shape=[jax.ShapeDtypeStruct((E, 64), F32),
                   jax.ShapeDtypeStruct((E, 128), F32)],
    )(gcat, ea, wec, be0, w_e1, be1, wla, bn0, w_n1, bn1)


def _node1(sums, x, w0x, w0a, b0, w1, b1, wpost):
    n = x.shape[0]
    bn = 2000 if n % 2000 == 0 else n

    def body(s_ref, x_ref, w0x_ref, w0a_ref, b0_ref, w1_ref, b1_ref,
             wp_ref, nf_ref, t2a_ref, t2bc_ref, rden_ref):
        stot = s_ref[0] + s_ref[1]
        rden = 1.0 / jnp.maximum(stot[:, 64:65], 1.0)
        agg = stot[:, 0:64] * rden
        h = jnp.maximum(
            jnp.dot(x_ref[...], w0x_ref[...], preferred_element_type=F32)
            + jnp.dot(agg, w0a_ref[...], preferred_element_type=F32)
            + b0_ref[...], 0.0)
        nf = jnp.maximum(
            jnp.dot(h, w1_ref[...], preferred_element_type=F32) + b1_ref[...], 0.0)
        nf_ref[...] = nf
        rden_ref[...] = rden
        t = jnp.dot(nf, wp_ref[...], preferred_element_type=F32)
        t2a_ref[...] = jnp.concatenate(
            [t[:, 0:64], jnp.zeros(t[:, 0:64].shape, F32)], axis=1)
        t2bc_ref[...] = t[:, 64:192]

    return pl.pallas_call(
        body,
        grid=(n // bn,),
        in_specs=[pl.BlockSpec((NC, bn, 128), lambda i: (0, i, 0)),
                  pl.BlockSpec((bn, 128), lambda i: (i, 0)),
                  _full(w0x.shape), _full(w0a.shape), _full(b0.shape),
                  _full(w1.shape), _full(b1.shape), _full(wpost.shape)],
        out_specs=[pl.BlockSpec((bn, 64), lambda i: (i, 0)),
                   pl.BlockSpec((bn, 128), lambda i: (i, 0)),
                   pl.BlockSpec((bn, 128), lambda i: (i, 0)),
                   pl.BlockSpec((bn, 1), lambda i: (i, 0))],
        out_shape=[jax.ShapeDtypeStruct((n, 64), F32),
                   jax.ShapeDtypeStruct((n, 128), F32),
                   jax.ShapeDtypeStruct((n, 128), F32),
                   jax.ShapeDtypeStruct((n, 1), F32)],
    )(sums, x, w0x, w0a, b0, w1, b1, wpost)


def _edge2(gcat, la, wla2, be0, w_e2, be1, wn2a, bn0, w_n2, bn1):
    E = gcat.shape[0]
    be = _edge_block(E)

    def body(g_ref, la_ref, wla2_ref, be0_ref, we2_ref, be1_ref,
             wn2a_ref, bn0_ref, wn2_ref, bn1_ref, m_ref):
        g = g_ref[...].astype(F32)
        h3 = jnp.maximum(
            g[:, 0:64]
            + jnp.dot(la_ref[...], wla2_ref[...], preferred_element_type=F32)
            + be0_ref[...], 0.0)
        la2 = jnp.dot(h3, we2_ref[...], preferred_element_type=F32) + be1_ref[...]
        h4 = jnp.maximum(
            g[:, 64:128]
            + jnp.dot(la2, wn2a_ref[...], preferred_element_type=F32)
            + bn0_ref[...], 0.0)
        m2 = jnp.dot(h4, wn2_ref[...], preferred_element_type=F32) + bn1_ref[...]
        m_ref[...] = jnp.concatenate([m2, jnp.zeros(m2.shape, F32)], axis=1)

    eb64 = pl.BlockSpec((be, 64), lambda i: (i, 0))
    eb128 = pl.BlockSpec((be, 128), lambda i: (i, 0))
    return pl.pallas_call(
        body,
        grid=(E // be,),
        in_specs=[eb128, eb64,
                  _full(wla2.shape), _full(be0.shape), _full(w_e2.shape),
                  _full(be1.shape), _full(wn2a.shape), _full(bn0.shape),
                  _full(w_n2.shape), _full(bn1.shape)],
        out_specs=eb128,
        out_shape=jax.ShapeDtypeStruct((E, 128), F32),
    )(gcat, la, wla2, be0, w_e2, be1, wn2a, bn0, w_n2, bn1)


def _node2(sums, rden, nf, w0n, w0a, b0, w1, b1):
    n = nf.shape[0]
    bn = 2000 if n % 2000 == 0 else n

    def body(s_ref, rd_ref, nf_ref, w0n_ref, w0a_ref, b0_ref,
             w1_ref, b1_ref, out_ref):
        stot = s_ref[0] + s_ref[1]
        agg = stot[:, 0:64] * rd_ref[...]
        h = jnp.maximum(
            jnp.dot(nf_ref[...], w0n_ref[...], preferred_element_type=F32)
            + jnp.dot(agg, w0a_ref[...], preferred_element_type=F32)
            + b0_ref[...], 0.0)
        out_ref[...] = jnp.dot(h, w1_ref[...], preferred_element_type=F32) + b1_ref[...]

    return pl.pallas_call(
        body,
        grid=(n // bn,),
        in_specs=[pl.BlockSpec((NC, bn, 128), lambda i: (0, i, 0)),
                  pl.BlockSpec((bn, 1), lambda i: (i, 0)),
                  pl.BlockSpec((bn, 64), lambda i: (i, 0)),
                  _full(w0n.shape), _full(w0a.shape), _full(b0.shape),
                  _full(w1.shape), _full(b1.shape)],
        out_specs=pl.BlockSpec((bn, 1), lambda i: (i, 0)),
        out_shape=jax.ShapeDtypeStruct((n, 1), F32),
    )(sums, rden, nf, w0n, w0a, b0, w1, b1)


def kernel(x, edge_index, edge_attr,
           e1_W0, e1_b0, e1_W1, e1_b1,
           n1a_W0, n1a_b0, n1a_W1, n1a_b1,
           n1u_W0, n1u_b0, n1u_W1, n1u_b1,
           e2_W0, e2_b0, e2_W1, e2_b1,
           n2a_W0, n2a_b0, n2a_W1, n2a_b1,
           n2u_W0, n2u_b0, n2u_W1, n2u_b1):
    n, d = x.shape
    E = edge_index.shape[1]
    npad = -(-n // 128) * 128
    row = edge_index[0]
    col = edge_index[1]
    eaT = edge_attr.T

    if E % (NW * 8) == 0 and E > 3 * 81920 and (E - 3 * 81920) % (NW * 8) == 0:
        parts = [81920, 81920, 81920, E - 3 * 81920]
    else:
        parts = [E]
    offs = [sum(parts[:i]) for i in range(len(parts))]

    rows = [lax.slice(row, (o,), (o + q,)) for o, q in zip(offs, parts)]
    cols = [lax.slice(col, (o,), (o + q,)) for o, q in zip(offs, parts)]
    row3s = []
    for r, q in zip(rows, parts):
        per = q // NW
        ch = _largest_chunk(per)
        row3s.append(r.reshape(NW, per // ch, ch))
    eaTs = [lax.slice(eaT, (0, o), (eaT.shape[0], o + q))
            for o, q in zip(offs, parts)]

    r1 = lambda b: b.reshape(1, -1)

    wcat1 = jnp.concatenate([e1_W0[:d], e1_W0[d:2 * d], n1a_W0[:d]], axis=1)
    ta, tbc = _node_pre(x, wcat1)

    e1_w = (e1_W0[2 * d:], r1(e1_b0), e1_W1, r1(e1_b1),
            n1a_W0[d:], r1(n1a_b0), n1a_W1, r1(n1a_b1))
    zero_part = jnp.zeros((NC, npad, 128), F32)

    gcats = [_sc_gather(ta, tbc, r, c) for r, c in zip(rows, cols)]
    las, mcats = [], []
    for g, ea in zip(gcats, eaTs):
        la_i, m_i = _edge1(g, ea, *e1_w)
        las.append(la_i)
        mcats.append(m_i)
    s1 = zero_part
    for m_i, r3 in zip(mcats, row3s):
        s1 = _sc_scatter([m_i], [r3], s1)

    wpost = jnp.concatenate([e2_W0[:64], e2_W0[64:128], n2a_W0[:64]], axis=1)
    nf, t2a, t2bc, rden = _node1(s1, x,
                                 n1u_W0[:d], n1u_W0[d:], r1(n1u_b0),
                                 n1u_W1, r1(n1u_b1), wpost)

    e2_w = (e2_W0[128:], r1(e2_b0), e2_W1, r1(e2_b1),
            n2a_W0[64:], r1(n2a_b0), n2a_W1, r1(n2a_b1))
    g2cats = [_sc_gather(t2a, t2bc, r, c) for r, c in zip(rows, cols)]
    m2cats = [_edge2(g, la_i, *e2_w)
              for g, la_i in zip(g2cats, las)]
    s2 = zero_part
    for m_i, r3 in zip(m2cats, row3s):
        s2 = _sc_scatter([m_i], [r3], s2)

    return _node2(s2, rden, nf,
                  n2u_W0[:64], n2u_W0[64:], r1(n2u_b0),
                  n2u_W1, r1(n2u_b1))


# speedup vs baseline: 1.2225x; 1.0411x over previous
import jax
import jax.numpy as jnp
from jax import lax
from jax.experimental import pallas as pl
from jax.experimental.pallas import tpu as pltpu
from jax.experimental.pallas import tpu_sc as plsc

NC = 2
NS = 16
NW = NC * NS

F32 = jnp.float32
BF16 = jnp.bfloat16

_SC_PARAMS = pltpu.CompilerParams(use_tc_tiling_on_sc=True)


def _largest_chunk(per):
    best = 8
    for c in range(8, 129, 8):
        if per % c == 0:
            best = c
    return best


def _edge_block(E):
    for b in range(4096, 7, -8):
        if E % b == 0:
            return b
    return E


def _sc_gather(ta, tbc, row, col):
    E = row.shape[0]
    per = E // NW
    ch = _largest_chunk(per)
    nchunks = per // ch

    mesh = plsc.VectorSubcoreMesh(core_axis_name="c", subcore_axis_name="s",
                                  num_cores=NC, num_subcores=NS)

    def body(ta_h, tbc_h, row_h, col_h, g_h,
             rowi, coli, ca, cb, sg0, sg1, sw0, sw1):
        c = lax.axis_index("c")
        s = lax.axis_index("s")
        wid = s * NC + c
        start = pl.multiple_of(wid * per, ch)
        pltpu.sync_copy(row_h.at[pl.ds(start, per)], rowi)
        pltpu.sync_copy(col_h.at[pl.ds(start, per)], coli)

        sg = (sg0, sg1)
        sw = (sw0, sw1)

        def issue(b, t, first):
            if not first:
                pltpu.make_async_copy(cb.at[b], g_h.at[pl.ds(start, ch)],
                                      sw[b]).wait()
            ro = rowi.at[pl.ds(t * ch, ch)]
            co = coli.at[pl.ds(t * ch, ch)]
            pltpu.async_copy(ta_h.at[ro], ca.at[b], sg[b])
            pltpu.async_copy(tbc_h.at[co], cb.at[b], sg[b])

        def finish(b, t):
            base = pl.multiple_of(start + t * ch, ch)
            ro = rowi.at[pl.ds(0, ch)]
            pltpu.make_async_copy(ta_h.at[ro], ca.at[b], sg[b]).wait()
            pltpu.make_async_copy(ta_h.at[ro], cb.at[b], sg[b]).wait()

            def addrow(r, carry):
                for k in range(4):
                    sl = pl.ds(16 * k, 16)
                    cb[b, r, sl] = cb[b, r, sl] + ca[b, r, sl]
                return carry

            lax.fori_loop(0, ch, addrow, 0, unroll=8)
            pltpu.async_copy(cb.at[b], g_h.at[pl.ds(base, ch)], sw[b])

        issue(0, 0, True)
        if nchunks > 1:
            issue(1, 1, True)

        def pair(tp, carry):
            t0 = 2 * tp
            t1 = t0 + 1
            finish(0, t0)
            issue(0, t0 + 2, False)
            finish(1, t1)

            @pl.when(t1 + 2 < nchunks)
            def _():
                issue(1, t1 + 2, False)

            return carry

        if nchunks % 2 == 1:
            lax.fori_loop(0, (nchunks - 1) // 2, pair, 0)
            finish(0, nchunks - 1)
        else:
            lax.fori_loop(0, (nchunks - 2) // 2, pair, 0)
            finish(0, nchunks - 2)
            finish(1, nchunks - 1)
        pltpu.make_async_copy(cb.at[0], g_h.at[pl.ds(start, ch)], sw[0]).wait()
        if nchunks > 1:
            pltpu.make_async_copy(cb.at[1], g_h.at[pl.ds(start, ch)],
                                  sw[1]).wait()

    call = pl.kernel(
        body,
        out_type=jax.ShapeDtypeStruct((E, 128), F32),
        mesh=mesh,
        scratch_types=[
            pltpu.VMEM((per,), jnp.int32),
            pltpu.VMEM((per,), jnp.int32),
            pltpu.VMEM((2, ch, 128), F32),
            pltpu.VMEM((2, ch, 128), F32),
            pltpu.SemaphoreType.DMA,
            pltpu.SemaphoreType.DMA,
            pltpu.SemaphoreType.DMA,
            pltpu.SemaphoreType.DMA,
        ],
        compiler_params=_SC_PARAMS,
    )
    return call(ta, tbc, row, col)


def _sc_scatter(vals_list, row3_list, prev):
    k = len(vals_list)
    infos = []
    for v, r3 in zip(vals_list, row3_list):
        infos.append((v.shape[0] // NW, r3.shape[2], r3.shape[1]))
    chmax = max(ch for _, ch, _ in infos)
    npad = prev.shape[1]
    rpt = npad // NS

    mesh = plsc.VectorSubcoreMesh(core_axis_name="c", subcore_axis_name="s",
                                  num_cores=NC, num_subcores=NS)

    def body(*refs):
        vals_hs = refs[0:k]
        row3_hs = refs[k:2 * k]
        prev_h = refs[2 * k]
        sum_h = refs[2 * k + 1]
        idxvs = refs[2 * k + 2:3 * k + 2]
        valvs = refs[3 * k + 2:4 * k + 2]
        ssum, sl0, sl1, ss0, ss1 = refs[4 * k + 2:]
        c = lax.axis_index("c")
        s = lax.axis_index("s")
        wid = s * NC + c
        soff = pl.multiple_of(s * rpt, 8)

        pltpu.sync_copy(prev_h.at[c, pl.ds(soff, rpt)],
                        ssum.at[pl.ds(soff, rpt)])
        plsc.subcore_barrier()

        sl = (sl0, sl1)
        ss = (ss0, ss1)

        for p in range(k):
            per, ch, nchunks = infos[p]
            vals_h = vals_hs[p]
            idxv = idxvs[p]
            valv = valvs[p]
            start = pl.multiple_of(wid * per, ch)
            pltpu.sync_copy(row3_hs[p].at[wid], idxv)

            def vbuf(b):
                return valv.at[b]

            def wait_scat(b):
                pltpu.make_async_copy(vbuf(b), ssum.at[idxv.at[0]],
                                      ss[b]).wait()

            def load(b, t, first):
                if not first:
                    wait_scat(b)
                pltpu.async_copy(
                    vals_h.at[pl.ds(pl.multiple_of(start + t * ch, ch), ch)],
                    vbuf(b), sl[b])

            def scat(b, t):
                pltpu.make_async_copy(vals_h.at[pl.ds(start, ch)],
                                      vbuf(b), sl[b]).wait()
                pltpu.async_copy(vbuf(b), ssum.at[idxv.at[t]], ss[b],
                                 add=True)

            load(0, 0, True)
            if nchunks > 1:
                load(1, 1, True)

            def pair(tp, carry):
                t0 = 2 * tp
                t1 = t0 + 1
                scat(0, t0)
                load(0, t0 + 2, False)
                scat(1, t1)

                @pl.when(t1 + 2 < nchunks)
                def _():
                    load(1, t1 + 2, False)

                return carry

            if nchunks % 2 == 1:
                lax.fori_loop(0, (nchunks - 1) // 2, pair, 0)
                scat(0, nchunks - 1)
            else:
                lax.fori_loop(0, (nchunks - 2) // 2, pair, 0)
                scat(0, nchunks - 2)
                scat(1, nchunks - 1)
            wait_scat(0)
            if nchunks > 1:
                wait_scat(1)
        plsc.subcore_barrier()

        pltpu.sync_copy(ssum.at[pl.ds(soff, rpt)],
                        sum_h.at[c, pl.ds(soff, rpt)])

    call = pl.kernel(
        body,
        out_type=jax.ShapeDtypeStruct((NC, npad, 128), F32),
        mesh=mesh,
        scratch_types=(
            [pltpu.VMEM((nch, ch), jnp.int32) for _, ch, nch in infos]
            + [pltpu.VMEM((2, ch, 128), F32) for _, ch, _ in infos] + [
                pltpu.VMEM_SHARED((npad, 128), F32),
                pltpu.SemaphoreType.DMA,
                pltpu.SemaphoreType.DMA,
                pltpu.SemaphoreType.DMA,
                pltpu.SemaphoreType.DMA,
            ]),
        compiler_params=_SC_PARAMS,
    )
    return call(*vals_list, *row3_list, prev)


def _full(shape):
    return pl.BlockSpec(shape, lambda i: (0,) * len(shape))


def _node_pre(x, wcat):
    n, d = x.shape
    bn = 2000 if n % 2000 == 0 else n

    def body(x_ref, w_ref, ta_ref, tbc_ref):
        t = jnp.dot(x_ref[...], w_ref[...], preferred_element_type=F32)
        ta_ref[...] = jnp.concatenate(
            [t[:, 0:64], jnp.zeros(t[:, 0:64].shape, F32)], axis=1)
        tbc_ref[...] = t[:, 64:192]

    o = jax.ShapeDtypeStruct((n, 128), F32)
    return pl.pallas_call(
        body,
        grid=(n // bn,),
        in_specs=[pl.BlockSpec((bn, d), lambda i: (i, 0)), _full(wcat.shape)],
        out_specs=[pl.BlockSpec((bn, 128), lambda i: (i, 0))] * 2,
        out_shape=[o, o],
    )(x, wcat)


def _edge1(gcat, ea, wec, be0, w_e1, be1, wla, bn0, w_n1, bn1):
    E = gcat.shape[0]
    be = _edge_block(E)

    def body(g_ref, ea_ref, wec_ref, be0_ref, we1_ref, be1_ref,
             wla_ref, bn0_ref, wn1_ref, bn1_ref, la_ref, m_ref):
        g = g_ref[...].astype(F32)
        eaw = lax.dot_general(ea_ref[...], wec_ref[...],
                              (((0,), (0,)), ((), ())),
                              preferred_element_type=F32)
        h1 = jnp.maximum(g[:, 0:64] + eaw + be0_ref[...], 0.0)
        la = jnp.dot(h1, we1_ref[...], preferred_element_type=F32) + be1_ref[...]
        la_ref[...] = la.astype(BF16)
        h2 = jnp.maximum(
            g[:, 64:128]
            + jnp.dot(la, wla_ref[...], preferred_element_type=F32)
            + bn0_ref[...], 0.0)
        m1 = jnp.dot(h2, wn1_ref[...], preferred_element_type=F32) + bn1_ref[...]
        lane = lax.broadcasted_iota(jnp.int32, (m1.shape[0], 64), 1)
        cnt = jnp.where(lane == 0, 1.0, 0.0).astype(F32)
        m_ref[...] = jnp.concatenate([m1, cnt], axis=1)

    eb64 = pl.BlockSpec((be, 64), lambda i: (i, 0))
    eb128 = pl.BlockSpec((be, 128), lambda i: (i, 0))
    return pl.pallas_call(
        body,
        grid=(E // be,),
        in_specs=[eb128, pl.BlockSpec((16, be), lambda i: (0, i)),
                  _full(wec.shape), _full(be0.shape), _full(w_e1.shape),
                  _full(be1.shape), _full(wla.shape), _full(bn0.shape),
                  _full(w_n1.shape), _full(bn1.shape)],
        out_specs=[eb64, eb128],
        out_shape=[jax.ShapeDtypeStruct((E, 64), BF16),
                   jax.ShapeDtypeStruct((E, 128), F32)],
    )(gcat, ea, wec, be0, w_e1, be1, wla, bn0, w_n1, bn1)


def _node1(sums, x, w0x, w0a, b0, w1, b1, wpost):
    n = x.shape[0]
    bn = 2000 if n % 2000 == 0 else n

    def body(s_ref, x_ref, w0x_ref, w0a_ref, b0_ref, w1_ref, b1_ref,
             wp_ref, nf_ref, t2a_ref, t2bc_ref, rden_ref):
        stot = s_ref[0] + s_ref[1]
        rden = 1.0 / jnp.maximum(stot[:, 64:65], 1.0)
        agg = stot[:, 0:64] * rden
        h = jnp.maximum(
            jnp.dot(x_ref[...], w0x_ref[...], preferred_element_type=F32)
            + jnp.dot(agg, w0a_ref[...], preferred_element_type=F32)
            + b0_ref[...], 0.0)
        nf = jnp.maximum(
            jnp.dot(h, w1_ref[...], preferred_element_type=F32) + b1_ref[...], 0.0)
        nf_ref[...] = nf
        rden_ref[...] = rden
        t = jnp.dot(nf, wp_ref[...], preferred_element_type=F32)
        t2a_ref[...] = jnp.concatenate(
            [t[:, 0:64], jnp.zeros(t[:, 0:64].shape, F32)], axis=1)
        t2bc_ref[...] = t[:, 64:192]

    return pl.pallas_call(
        body,
        grid=(n // bn,),
        in_specs=[pl.BlockSpec((NC, bn, 128), lambda i: (0, i, 0)),
                  pl.BlockSpec((bn, 128), lambda i: (i, 0)),
                  _full(w0x.shape), _full(w0a.shape), _full(b0.shape),
                  _full(w1.shape), _full(b1.shape), _full(wpost.shape)],
        out_specs=[pl.BlockSpec((bn, 64), lambda i: (i, 0)),
                   pl.BlockSpec((bn, 128), lambda i: (i, 0)),
                   pl.BlockSpec((bn, 128), lambda i: (i, 0)),
                   pl.BlockSpec((bn, 1), lambda i: (i, 0))],
        out_shape=[jax.ShapeDtypeStruct((n, 64), F32),
                   jax.ShapeDtypeStruct((n, 128), F32),
                   jax.ShapeDtypeStruct((n, 128), F32),
                   jax.ShapeDtypeStruct((n, 1), F32)],
    )(sums, x, w0x, w0a, b0, w1, b1, wpost)


def _edge2(gcat, la, wla2, be0, w_e2, be1, wn2a, bn0, w_n2, bn1):
    E = gcat.shape[0]
    be = _edge_block(E)

    def body(g_ref, la_ref, wla2_ref, be0_ref, we2_ref, be1_ref,
             wn2a_ref, bn0_ref, wn2_ref, bn1_ref, m_ref):
        g = g_ref[...].astype(F32)
        h3 = jnp.maximum(
            g[:, 0:64]
            + jnp.dot(la_ref[...].astype(F32), wla2_ref[...],
                      preferred_element_type=F32)
            + be0_ref[...], 0.0)
        la2 = jnp.dot(h3, we2_ref[...], preferred_element_type=F32) + be1_ref[...]
        h4 = jnp.maximum(
            g[:, 64:128]
            + jnp.dot(la2, wn2a_ref[...], preferred_element_type=F32)
            + bn0_ref[...], 0.0)
        m2 = jnp.dot(h4, wn2_ref[...], preferred_element_type=F32) + bn1_ref[...]
        m_ref[...] = jnp.concatenate([m2, jnp.zeros(m2.shape, F32)], axis=1)

    eb64 = pl.BlockSpec((be, 64), lambda i: (i, 0))
    eb128 = pl.BlockSpec((be, 128), lambda i: (i, 0))
    return pl.pallas_call(
        body,
        grid=(E // be,),
        in_specs=[eb128, eb64,
                  _full(wla2.shape), _full(be0.shape), _full(w_e2.shape),
                  _full(be1.shape), _full(wn2a.shape), _full(bn0.shape),
                  _full(w_n2.shape), _full(bn1.shape)],
        out_specs=eb128,
        out_shape=jax.ShapeDtypeStruct((E, 128), F32),
    )(gcat, la, wla2, be0, w_e2, be1, wn2a, bn0, w_n2, bn1)


def _node2(sums, rden, nf, w0n, w0a, b0, w1, b1):
    n = nf.shape[0]
    bn = 2000 if n % 2000 == 0 else n

    def body(s_ref, rd_ref, nf_ref, w0n_ref, w0a_ref, b0_ref,
             w1_ref, b1_ref, out_ref):
        stot = s_ref[0] + s_ref[1]
        agg = stot[:, 0:64] * rd_ref[...]
        h = jnp.maximum(
            jnp.dot(nf_ref[...], w0n_ref[...], preferred_element_type=F32)
            + jnp.dot(agg, w0a_ref[...], preferred_element_type=F32)
            + b0_ref[...], 0.0)
        out_ref[...] = jnp.dot(h, w1_ref[...], preferred_element_type=F32) + b1_ref[...]

    return pl.pallas_call(
        body,
        grid=(n // bn,),
        in_specs=[pl.BlockSpec((NC, bn, 128), lambda i: (0, i, 0)),
                  pl.BlockSpec((bn, 1), lambda i: (i, 0)),
                  pl.BlockSpec((bn, 64), lambda i: (i, 0)),
                  _full(w0n.shape), _full(w0a.shape), _full(b0.shape),
                  _full(w1.shape), _full(b1.shape)],
        out_specs=pl.BlockSpec((bn, 1), lambda i: (i, 0)),
        out_shape=jax.ShapeDtypeStruct((n, 1), F32),
    )(sums, rden, nf, w0n, w0a, b0, w1, b1)


def kernel(x, edge_index, edge_attr,
           e1_W0, e1_b0, e1_W1, e1_b1,
           n1a_W0, n1a_b0, n1a_W1, n1a_b1,
           n1u_W0, n1u_b0, n1u_W1, n1u_b1,
           e2_W0, e2_b0, e2_W1, e2_b1,
           n2a_W0, n2a_b0, n2a_W1, n2a_b1,
           n2u_W0, n2u_b0, n2u_W1, n2u_b1):
    n, d = x.shape
    E = edge_index.shape[1]
    npad = -(-n // 128) * 128
    row = edge_index[0]
    col = edge_index[1]
    eaT = edge_attr.T

    if E % (NW * 8) == 0 and E > 3 * 81920 and (E - 3 * 81920) % (NW * 8) == 0:
        parts = [81920, 81920, 81920, E - 3 * 81920]
    else:
        parts = [E]
    offs = [sum(parts[:i]) for i in range(len(parts))]

    rows = [lax.slice(row, (o,), (o + q,)) for o, q in zip(offs, parts)]
    cols = [lax.slice(col, (o,), (o + q,)) for o, q in zip(offs, parts)]
    row3s = []
    for r, q in zip(rows, parts):
        per = q // NW
        ch = _largest_chunk(per)
        row3s.append(r.reshape(NW, per // ch, ch))
    eaTs = [lax.slice(eaT, (0, o), (eaT.shape[0], o + q))
            for o, q in zip(offs, parts)]

    r1 = lambda b: b.reshape(1, -1)

    wcat1 = jnp.concatenate([e1_W0[:d], e1_W0[d:2 * d], n1a_W0[:d]], axis=1)
    ta, tbc = _node_pre(x, wcat1)

    e1_w = (e1_W0[2 * d:], r1(e1_b0), e1_W1, r1(e1_b1),
            n1a_W0[d:], r1(n1a_b0), n1a_W1, r1(n1a_b1))
    zero_part = jnp.zeros((NC, npad, 128), F32)

    gcats = [_sc_gather(ta, tbc, r, c) for r, c in zip(rows, cols)]
    las, mcats = [], []
    for g, ea in zip(gcats, eaTs):
        la_i, m_i = _edge1(g, ea, *e1_w)
        las.append(la_i)
        mcats.append(m_i)
    s1 = zero_part
    for m_i, r3 in zip(mcats, row3s):
        s1 = _sc_scatter([m_i], [r3], s1)

    wpost = jnp.concatenate([e2_W0[:64], e2_W0[64:128], n2a_W0[:64]], axis=1)
    nf, t2a, t2bc, rden = _node1(s1, x,
                                 n1u_W0[:d], n1u_W0[d:], r1(n1u_b0),
                                 n1u_W1, r1(n1u_b1), wpost)

    e2_w = (e2_W0[128:], r1(e2_b0), e2_W1, r1(e2_b1),
            n2a_W0[64:], r1(n2a_b0), n2a_W1, r1(n2a_b1))
    g2cats = [_sc_gather(t2a, t2bc, r, c) for r, c in zip(rows, cols)]
    m2cats = [_edge2(g, la_i, *e2_w)
              for g, la_i in zip(g2cats, las)]
    s2 = zero_part
    for m_i, r3 in zip(m2cats, row3s):
        s2 = _sc_scatter([m_i], [r3], s2)

    return _node2(s2, rden, nf,
                  n2u_W0[:64], n2u_W0[64:], r1(n2u_b0),
                  n2u_W1, r1(n2u_b1))


# speedup vs baseline: 1.2258x; 1.0027x over previous
import jax
import jax.numpy as jnp
from jax import lax
from jax.experimental import pallas as pl
from jax.experimental.pallas import tpu as pltpu
from jax.experimental.pallas import tpu_sc as plsc

NC = 2
NS = 16
NW = NC * NS

F32 = jnp.float32
BF16 = jnp.bfloat16

_SC_PARAMS = pltpu.CompilerParams(use_tc_tiling_on_sc=True)


def _largest_chunk(per):
    best = 8
    for c in range(8, 129, 8):
        if per % c == 0:
            best = c
    return best


def _edge_block(E):
    for b in range(4096, 7, -8):
        if E % b == 0:
            return b
    return E


def _sc_gather(ta, tbc, row, col):
    E = row.shape[0]
    per = E // NW
    ch = _largest_chunk(per)
    nchunks = per // ch

    mesh = plsc.VectorSubcoreMesh(core_axis_name="c", subcore_axis_name="s",
                                  num_cores=NC, num_subcores=NS)

    def body(ta_h, tbc_h, row_h, col_h, g_h,
             rowi, coli, ca, cb, sg0, sg1, sw0, sw1):
        c = lax.axis_index("c")
        s = lax.axis_index("s")
        wid = s * NC + c
        start = pl.multiple_of(wid * per, ch)
        pltpu.sync_copy(row_h.at[pl.ds(start, per)], rowi)
        pltpu.sync_copy(col_h.at[pl.ds(start, per)], coli)

        sg = (sg0, sg1)
        sw = (sw0, sw1)

        def issue(b, t, first):
            if not first:
                pltpu.make_async_copy(cb.at[b], g_h.at[pl.ds(start, ch)],
                                      sw[b]).wait()
            ro = rowi.at[pl.ds(t * ch, ch)]
            co = coli.at[pl.ds(t * ch, ch)]
            pltpu.async_copy(ta_h.at[ro], ca.at[b], sg[b])
            pltpu.async_copy(tbc_h.at[co], cb.at[b], sg[b])

        def finish(b, t):
            base = pl.multiple_of(start + t * ch, ch)
            ro = rowi.at[pl.ds(0, ch)]
            pltpu.make_async_copy(ta_h.at[ro], ca.at[b], sg[b]).wait()
            pltpu.make_async_copy(ta_h.at[ro], cb.at[b], sg[b]).wait()

            def addrow(r, carry):
                for k in range(4):
                    sl = pl.ds(16 * k, 16)
                    cb[b, r, sl] = cb[b, r, sl] + ca[b, r, sl]
                return carry

            lax.fori_loop(0, ch, addrow, 0, unroll=8)
            pltpu.async_copy(cb.at[b], g_h.at[pl.ds(base, ch)], sw[b])

        issue(0, 0, True)
        if nchunks > 1:
            issue(1, 1, True)

        def pair(tp, carry):
            t0 = 2 * tp
            t1 = t0 + 1
            finish(0, t0)
            issue(0, t0 + 2, False)
            finish(1, t1)

            @pl.when(t1 + 2 < nchunks)
            def _():
                issue(1, t1 + 2, False)

            return carry

        if nchunks % 2 == 1:
            lax.fori_loop(0, (nchunks - 1) // 2, pair, 0)
            finish(0, nchunks - 1)
        else:
            lax.fori_loop(0, (nchunks - 2) // 2, pair, 0)
            finish(0, nchunks - 2)
            finish(1, nchunks - 1)
        pltpu.make_async_copy(cb.at[0], g_h.at[pl.ds(start, ch)], sw[0]).wait()
        if nchunks > 1:
            pltpu.make_async_copy(cb.at[1], g_h.at[pl.ds(start, ch)],
                                  sw[1]).wait()

    call = pl.kernel(
        body,
        out_type=jax.ShapeDtypeStruct((E, 128), F32),
        mesh=mesh,
        scratch_types=[
            pltpu.VMEM((per,), jnp.int32),
            pltpu.VMEM((per,), jnp.int32),
            pltpu.VMEM((2, ch, 128), F32),
            pltpu.VMEM((2, ch, 128), F32),
            pltpu.SemaphoreType.DMA,
            pltpu.SemaphoreType.DMA,
            pltpu.SemaphoreType.DMA,
            pltpu.SemaphoreType.DMA,
        ],
        compiler_params=_SC_PARAMS,
    )
    return call(ta, tbc, row, col)


def _sc_scatter(vals_list, row3_list, prev):
    k = len(vals_list)
    w = vals_list[0].shape[1]
    infos = []
    for v, r3 in zip(vals_list, row3_list):
        infos.append((v.shape[0] // NW, r3.shape[2], r3.shape[1]))
    npad = prev.shape[1]
    rpt = npad // NS

    mesh = plsc.VectorSubcoreMesh(core_axis_name="c", subcore_axis_name="s",
                                  num_cores=NC, num_subcores=NS)

    def body(*refs):
        vals_hs = refs[0:k]
        row3_hs = refs[k:2 * k]
        prev_h = refs[2 * k]
        sum_h = refs[2 * k + 1]
        idxvs = refs[2 * k + 2:3 * k + 2]
        valvs = refs[3 * k + 2:4 * k + 2]
        ssum, sl0, sl1, ss0, ss1 = refs[4 * k + 2:]
        c = lax.axis_index("c")
        s = lax.axis_index("s")
        wid = s * NC + c
        soff = pl.multiple_of(s * rpt, 8)

        pltpu.sync_copy(prev_h.at[c, pl.ds(soff, rpt)],
                        ssum.at[pl.ds(soff, rpt)])
        plsc.subcore_barrier()

        sl = (sl0, sl1)
        ss = (ss0, ss1)

        for p in range(k):
            per, ch, nchunks = infos[p]
            vals_h = vals_hs[p]
            idxv = idxvs[p]
            valv = valvs[p]
            start = pl.multiple_of(wid * per, ch)
            pltpu.sync_copy(row3_hs[p].at[wid], idxv)

            def vbuf(b):
                return valv.at[b]

            def wait_scat(b):
                pltpu.make_async_copy(vbuf(b), ssum.at[idxv.at[0]],
                                      ss[b]).wait()

            def load(b, t, first):
                if not first:
                    wait_scat(b)
                pltpu.async_copy(
                    vals_h.at[pl.ds(pl.multiple_of(start + t * ch, ch), ch)],
                    vbuf(b), sl[b])

            def scat(b, t):
                pltpu.make_async_copy(vals_h.at[pl.ds(start, ch)],
                                      vbuf(b), sl[b]).wait()
                pltpu.async_copy(vbuf(b), ssum.at[idxv.at[t]], ss[b],
                                 add=True)

            load(0, 0, True)
            if nchunks > 1:
                load(1, 1, True)

            def pair(tp, carry):
                t0 = 2 * tp
                t1 = t0 + 1
                scat(0, t0)
                load(0, t0 + 2, False)
                scat(1, t1)

                @pl.when(t1 + 2 < nchunks)
                def _():
                    load(1, t1 + 2, False)

                return carry

            if nchunks % 2 == 1:
                lax.fori_loop(0, (nchunks - 1) // 2, pair, 0)
                scat(0, nchunks - 1)
            else:
                lax.fori_loop(0, (nchunks - 2) // 2, pair, 0)
                scat(0, nchunks - 2)
                scat(1, nchunks - 1)
            wait_scat(0)
            if nchunks > 1:
                wait_scat(1)
        plsc.subcore_barrier()

        pltpu.sync_copy(ssum.at[pl.ds(soff, rpt)],
                        sum_h.at[c, pl.ds(soff, rpt)])

    call = pl.kernel(
        body,
        out_type=jax.ShapeDtypeStruct((NC, npad, w), F32),
        mesh=mesh,
        scratch_types=(
            [pltpu.VMEM((nch, ch), jnp.int32) for _, ch, nch in infos]
            + [pltpu.VMEM((2, ch, w), F32) for _, ch, _ in infos] + [
                pltpu.VMEM_SHARED((npad, w), F32),
                pltpu.SemaphoreType.DMA,
                pltpu.SemaphoreType.DMA,
                pltpu.SemaphoreType.DMA,
                pltpu.SemaphoreType.DMA,
            ]),
        compiler_params=_SC_PARAMS,
    )
    return call(*vals_list, *row3_list, prev)


def _full(shape):
    return pl.BlockSpec(shape, lambda i: (0,) * len(shape))


def _node_pre(x, wcat):
    n, d = x.shape
    bn = 2000 if n % 2000 == 0 else n

    def body(x_ref, w_ref, ta_ref, tbc_ref):
        t = jnp.dot(x_ref[...], w_ref[...], preferred_element_type=F32)
        ta_ref[...] = jnp.concatenate(
            [t[:, 0:64], jnp.zeros(t[:, 0:64].shape, F32)], axis=1)
        tbc_ref[...] = t[:, 64:192]

    o = jax.ShapeDtypeStruct((n, 128), F32)
    return pl.pallas_call(
        body,
        grid=(n // bn,),
        in_specs=[pl.BlockSpec((bn, d), lambda i: (i, 0)), _full(wcat.shape)],
        out_specs=[pl.BlockSpec((bn, 128), lambda i: (i, 0))] * 2,
        out_shape=[o, o],
    )(x, wcat)


def _edge1(gcat, ea, wec, be0, w_e1, be1, wla, bn0, w_n1, bn1):
    E = gcat.shape[0]
    be = _edge_block(E)

    def body(g_ref, ea_ref, wec_ref, be0_ref, we1_ref, be1_ref,
             wla_ref, bn0_ref, wn1_ref, bn1_ref, la_ref, m_ref):
        g = g_ref[...].astype(F32)
        eaw = lax.dot_general(ea_ref[...], wec_ref[...],
                              (((0,), (0,)), ((), ())),
                              preferred_element_type=F32)
        h1 = jnp.maximum(g[:, 0:64] + eaw + be0_ref[...], 0.0)
        la = jnp.dot(h1, we1_ref[...], preferred_element_type=F32) + be1_ref[...]
        la_ref[...] = la.astype(BF16)
        h2 = jnp.maximum(
            g[:, 64:128]
            + jnp.dot(la, wla_ref[...], preferred_element_type=F32)
            + bn0_ref[...], 0.0)
        m1 = jnp.dot(h2, wn1_ref[...], preferred_element_type=F32) + bn1_ref[...]
        lane = lax.broadcasted_iota(jnp.int32, (m1.shape[0], 64), 1)
        cnt = jnp.where(lane == 0, 1.0, 0.0).astype(F32)
        m_ref[...] = jnp.concatenate([m1, cnt], axis=1)

    eb64 = pl.BlockSpec((be, 64), lambda i: (i, 0))
    eb128 = pl.BlockSpec((be, 128), lambda i: (i, 0))
    return pl.pallas_call(
        body,
        grid=(E // be,),
        in_specs=[eb128, pl.BlockSpec((16, be), lambda i: (0, i)),
                  _full(wec.shape), _full(be0.shape), _full(w_e1.shape),
                  _full(be1.shape), _full(wla.shape), _full(bn0.shape),
                  _full(w_n1.shape), _full(bn1.shape)],
        out_specs=[eb64, eb128],
        out_shape=[jax.ShapeDtypeStruct((E, 64), BF16),
                   jax.ShapeDtypeStruct((E, 128), F32)],
    )(gcat, ea, wec, be0, w_e1, be1, wla, bn0, w_n1, bn1)


def _node1(sums, x, w0x, w0a, b0, w1, b1, wpost):
    n = x.shape[0]
    bn = 2000 if n % 2000 == 0 else n

    def body(s_ref, x_ref, w0x_ref, w0a_ref, b0_ref, w1_ref, b1_ref,
             wp_ref, nf_ref, t2a_ref, t2bc_ref, rden_ref):
        stot = s_ref[0] + s_ref[1]
        rden = 1.0 / jnp.maximum(stot[:, 64:65], 1.0)
        agg = stot[:, 0:64] * rden
        h = jnp.maximum(
            jnp.dot(x_ref[...], w0x_ref[...], preferred_element_type=F32)
            + jnp.dot(agg, w0a_ref[...], preferred_element_type=F32)
            + b0_ref[...], 0.0)
        nf = jnp.maximum(
            jnp.dot(h, w1_ref[...], preferred_element_type=F32) + b1_ref[...], 0.0)
        nf_ref[...] = nf
        rden_ref[...] = rden
        t = jnp.dot(nf, wp_ref[...], preferred_element_type=F32)
        t2a_ref[...] = jnp.concatenate(
            [t[:, 0:64], jnp.zeros(t[:, 0:64].shape, F32)], axis=1)
        t2bc_ref[...] = t[:, 64:192]

    return pl.pallas_call(
        body,
        grid=(n // bn,),
        in_specs=[pl.BlockSpec((NC, bn, 128), lambda i: (0, i, 0)),
                  pl.BlockSpec((bn, 128), lambda i: (i, 0)),
                  _full(w0x.shape), _full(w0a.shape), _full(b0.shape),
                  _full(w1.shape), _full(b1.shape), _full(wpost.shape)],
        out_specs=[pl.BlockSpec((bn, 64), lambda i: (i, 0)),
                   pl.BlockSpec((bn, 128), lambda i: (i, 0)),
                   pl.BlockSpec((bn, 128), lambda i: (i, 0)),
                   pl.BlockSpec((bn, 1), lambda i: (i, 0))],
        out_shape=[jax.ShapeDtypeStruct((n, 64), F32),
                   jax.ShapeDtypeStruct((n, 128), F32),
                   jax.ShapeDtypeStruct((n, 128), F32),
                   jax.ShapeDtypeStruct((n, 1), F32)],
    )(sums, x, w0x, w0a, b0, w1, b1, wpost)


def _edge2(gcat, la, wla2, be0, w_e2, be1, wn2a, bn0, w_n2, bn1):
    E = gcat.shape[0]
    be = _edge_block(E)

    def body(g_ref, la_ref, wla2_ref, be0_ref, we2_ref, be1_ref,
             wn2a_ref, bn0_ref, wn2_ref, bn1_ref, m_ref):
        g = g_ref[...].astype(F32)
        h3 = jnp.maximum(
            g[:, 0:64]
            + jnp.dot(la_ref[...].astype(F32), wla2_ref[...],
                      preferred_element_type=F32)
            + be0_ref[...], 0.0)
        la2 = jnp.dot(h3, we2_ref[...], preferred_element_type=F32) + be1_ref[...]
        h4 = jnp.maximum(
            g[:, 64:128]
            + jnp.dot(la2, wn2a_ref[...], preferred_element_type=F32)
            + bn0_ref[...], 0.0)
        m_ref[...] = jnp.dot(h4, wn2_ref[...], preferred_element_type=F32) + bn1_ref[...]

    eb64 = pl.BlockSpec((be, 64), lambda i: (i, 0))
    eb128 = pl.BlockSpec((be, 128), lambda i: (i, 0))
    return pl.pallas_call(
        body,
        grid=(E // be,),
        in_specs=[eb128, eb64,
                  _full(wla2.shape), _full(be0.shape), _full(w_e2.shape),
                  _full(be1.shape), _full(wn2a.shape), _full(bn0.shape),
                  _full(w_n2.shape), _full(bn1.shape)],
        out_specs=eb64,
        out_shape=jax.ShapeDtypeStruct((E, 64), F32),
    )(gcat, la, wla2, be0, w_e2, be1, wn2a, bn0, w_n2, bn1)


def _node2(sums, rden, nf, w0n, w0a, b0, w1, b1):
    n = nf.shape[0]
    bn = 2000 if n % 2000 == 0 else n

    def body(s_ref, rd_ref, nf_ref, w0n_ref, w0a_ref, b0_ref,
             w1_ref, b1_ref, out_ref):
        stot = s_ref[0] + s_ref[1]
        agg = stot * rd_ref[...]
        h = jnp.maximum(
            jnp.dot(nf_ref[...], w0n_ref[...], preferred_element_type=F32)
            + jnp.dot(agg, w0a_ref[...], preferred_element_type=F32)
            + b0_ref[...], 0.0)
        out_ref[...] = jnp.dot(h, w1_ref[...], preferred_element_type=F32) + b1_ref[...]

    return pl.pallas_call(
        body,
        grid=(n // bn,),
        in_specs=[pl.BlockSpec((NC, bn, 64), lambda i: (0, i, 0)),
                  pl.BlockSpec((bn, 1), lambda i: (i, 0)),
                  pl.BlockSpec((bn, 64), lambda i: (i, 0)),
                  _full(w0n.shape), _full(w0a.shape), _full(b0.shape),
                  _full(w1.shape), _full(b1.shape)],
        out_specs=pl.BlockSpec((bn, 1), lambda i: (i, 0)),
        out_shape=jax.ShapeDtypeStruct((n, 1), F32),
    )(sums, rden, nf, w0n, w0a, b0, w1, b1)


def kernel(x, edge_index, edge_attr,
           e1_W0, e1_b0, e1_W1, e1_b1,
           n1a_W0, n1a_b0, n1a_W1, n1a_b1,
           n1u_W0, n1u_b0, n1u_W1, n1u_b1,
           e2_W0, e2_b0, e2_W1, e2_b1,
           n2a_W0, n2a_b0, n2a_W1, n2a_b1,
           n2u_W0, n2u_b0, n2u_W1, n2u_b1):
    n, d = x.shape
    E = edge_index.shape[1]
    npad = -(-n // 128) * 128
    row = edge_index[0]
    col = edge_index[1]
    eaT = edge_attr.T

    if E % (NW * 8) == 0 and E > 3 * 81920 and (E - 3 * 81920) % (NW * 8) == 0:
        parts = [81920, 81920, 81920, E - 3 * 81920]
    else:
        parts = [E]
    offs = [sum(parts[:i]) for i in range(len(parts))]

    rows = [lax.slice(row, (o,), (o + q,)) for o, q in zip(offs, parts)]
    cols = [lax.slice(col, (o,), (o + q,)) for o, q in zip(offs, parts)]
    row3s = []
    for r, q in zip(rows, parts):
        per = q // NW
        ch = _largest_chunk(per)
        row3s.append(r.reshape(NW, per // ch, ch))
    eaTs = [lax.slice(eaT, (0, o), (eaT.shape[0], o + q))
            for o, q in zip(offs, parts)]

    r1 = lambda b: b.reshape(1, -1)

    wcat1 = jnp.concatenate([e1_W0[:d], e1_W0[d:2 * d], n1a_W0[:d]], axis=1)
    ta, tbc = _node_pre(x, wcat1)

    e1_w = (e1_W0[2 * d:], r1(e1_b0), e1_W1, r1(e1_b1),
            n1a_W0[d:], r1(n1a_b0), n1a_W1, r1(n1a_b1))
    zero_part = jnp.zeros((NC, npad, 128), F32)

    gcats = [_sc_gather(ta, tbc, r, c) for r, c in zip(rows, cols)]
    las, mcats = [], []
    for g, ea in zip(gcats, eaTs):
        la_i, m_i = _edge1(g, ea, *e1_w)
        las.append(la_i)
        mcats.append(m_i)
    s1 = zero_part
    for m_i, r3 in zip(mcats, row3s):
        s1 = _sc_scatter([m_i], [r3], s1)

    wpost = jnp.concatenate([e2_W0[:64], e2_W0[64:128], n2a_W0[:64]], axis=1)
    nf, t2a, t2bc, rden = _node1(s1, x,
                                 n1u_W0[:d], n1u_W0[d:], r1(n1u_b0),
                                 n1u_W1, r1(n1u_b1), wpost)

    e2_w = (e2_W0[128:], r1(e2_b0), e2_W1, r1(e2_b1),
            n2a_W0[64:], r1(n2a_b0), n2a_W1, r1(n2a_b1))
    g2cats = [_sc_gather(t2a, t2bc, r, c) for r, c in zip(rows, cols)]
    m2cats = [_edge2(g, la_i, *e2_w)
              for g, la_i in zip(g2cats, las)]
    s2 = jnp.zeros((NC, npad, 64), F32)
    for m_i, r3 in zip(m2cats, row3s):
        s2 = _sc_scatter([m_i], [r3], s2)

    return _node2(s2, rden, nf,
                  n2u_W0[:64], n2u_W0[64:], r1(n2u_b0),
                  n2u_W1, r1(n2u_b1))
